# Initial kernel scaffold; baseline (speedup 1.0000x reference)
import jax
import jax.numpy as jnp
from jax.experimental import pallas as pl


def kernel(x, edge_index, params):
    raise NotImplementedError("write your pallas kernel here")


# speedup vs baseline: 36.1953x; 36.1953x over previous
import functools

import jax
import jax.numpy as jnp
from jax import lax
from jax.experimental import pallas as pl
from jax.experimental.pallas import tpu as pltpu
from jax.experimental.pallas import tpu_sc as plsc

N = 10000
E = 320000
HID = 128
HEADS = 8
HD = 16

NC = 2
NS = 16
NW = NC * NS
EB = E + N
CHUNK = 64
STEPS = -(-EB // (NW * CHUNK))
EP = NW * CHUNK * STEPS
ROWPAD = N + 112
PAD_IDX = N

BLK = 400


def _mesh():
  return plsc.VectorSubcoreMesh(core_axis_name="c", subcore_axis_name="s")


def _sc_gat_edges(xl_pad, xr_pad, s1d, d1d, att, z128):
  @functools.partial(
      pl.kernel,
      out_type=[
          jax.ShapeDtypeStruct((NC, ROWPAD, HID), jnp.float32),
          jax.ShapeDtypeStruct((EP, HD), jnp.float32),
      ],
      mesh=_mesh(),
      scratch_types=[
          pltpu.VMEM((HEADS, HD), jnp.float32),
          pltpu.VMEM((1, CHUNK), jnp.int32),
          pltpu.VMEM((1, CHUNK), jnp.int32),
          pltpu.VMEM((CHUNK, HID), jnp.float32),
          pltpu.VMEM((CHUNK, HID), jnp.float32),
          pltpu.VMEM((CHUNK, HD), jnp.float32),
          pltpu.VMEM_SHARED((ROWPAD, HID), jnp.float32),
          pltpu.SemaphoreType.DMA,
      ],
  )
  def k(xl_h, xr_h, s_h, d_h, att_h, z128_h,
        acc_out, ex_out,
        att_v, sidx, didx, xjb, xib, exb, acc_s, sem):
    cid = lax.axis_index("c")
    sid = lax.axis_index("s")
    wid = sid * NC + cid

    zrows = ROWPAD // NS
    pltpu.sync_copy(z128_h.at[pl.ds(0, CHUNK)], xjb)
    for q in range(10):
      rows = CHUNK if q < 9 else zrows - 9 * CHUNK
      off = sid * zrows + q * CHUNK
      pltpu.sync_copy(xjb.at[pl.ds(0, rows)], acc_s.at[pl.ds(off, rows)])
    pltpu.sync_copy(att_h, att_v)
    plsc.subcore_barrier()

    att_rows = [att_v[h, :] for h in range(HEADS)]
    lo8 = lax.iota(jnp.int32, 16) < 8
    base_e = wid * (STEPS * CHUNK)

    def step_body(t, carry):
      e0 = base_e + t * CHUNK
      pltpu.sync_copy(s_h.at[pl.ds(e0, CHUNK)], sidx.at[0])
      pltpu.sync_copy(d_h.at[pl.ds(e0, CHUNK)], didx.at[0])
      cp1 = pltpu.async_copy(xl_h.at[sidx.at[0]], xjb, sem)
      cp2 = pltpu.async_copy(xr_h.at[didx.at[0]], xib, sem)
      cp1.wait()
      cp2.wait()

      def edge_body(e, c2):
        bs = []
        acc = None
        for j in range(HEADS):
          sl = pl.ds(j * HD, HD)
          b = xjb[e, sl]
          bs.append(b)
          th = xib[e, sl] + b
          th = jnp.maximum(th, th * 0.2)
          pj = th * att_rows[j]
          acc = pj if acc is None else acc + pj
        l = acc + lax.rev(acc, (0,))
        lvec = jnp.where(lo8, l, 0.0)
        exv = jnp.exp(lvec)
        exb[e, :] = exv
        exlo = jnp.where(lo8, exv, 0.0)
        ex8 = exlo + lax.rev(exlo, (0,))
        for j in range(HEADS):
          sl = pl.ds(j * HD, HD)
          xjb[e, sl] = bs[j] * ex8
        return c2

      lax.fori_loop(0, CHUNK, edge_body, 0)

      pltpu.sync_copy(xjb, acc_s.at[didx.at[0]], add=True)
      pltpu.sync_copy(exb, ex_out.at[pl.ds(e0, CHUNK)])
      return carry

    lax.fori_loop(0, STEPS, step_body, 0)
    plsc.subcore_barrier()

    for q in range(10):
      rows = CHUNK if q < 9 else zrows - 9 * CHUNK
      off = sid * zrows + q * CHUNK
      pltpu.sync_copy(acc_s.at[pl.ds(off, rows)], xjb.at[pl.ds(0, rows)])
      pltpu.sync_copy(xjb.at[pl.ds(0, rows)], acc_out.at[cid].at[pl.ds(off, rows)])

  return k(xl_pad, xr_pad, s1d, d1d, att, z128)


def _sc_gat_denom(d1d, exv, z128):
  @functools.partial(
      pl.kernel,
      out_type=[jax.ShapeDtypeStruct((NC, ROWPAD, HID), jnp.float32)],
      mesh=_mesh(),
      scratch_types=[
          pltpu.VMEM((1, CHUNK), jnp.int32),
          pltpu.VMEM((CHUNK, HD), jnp.float32),
          pltpu.VMEM((CHUNK, HID), jnp.float32),
          pltpu.VMEM_SHARED((ROWPAD, HID), jnp.float32),
          pltpu.SemaphoreType.DMA,
      ],
  )
  def k(d_h, ex_h, z128_h, den_out, didx, exb, exw, den_s, sem):
    cid = lax.axis_index("c")
    sid = lax.axis_index("s")
    wid = sid * NC + cid
    zrows = ROWPAD // NS
    pltpu.sync_copy(z128_h.at[pl.ds(0, CHUNK)], exw)
    for q in range(10):
      rows = CHUNK if q < 9 else zrows - 9 * CHUNK
      off = sid * zrows + q * CHUNK
      pltpu.sync_copy(exw.at[pl.ds(0, rows)], den_s.at[pl.ds(off, rows)])
    plsc.subcore_barrier()

    base_e = wid * (STEPS * CHUNK)

    def step_body(t, carry):
      e0 = base_e + t * CHUNK
      pltpu.sync_copy(d_h.at[pl.ds(e0, CHUNK)], didx.at[0])
      pltpu.sync_copy(ex_h.at[pl.ds(e0, CHUNK)], exb)

      def edge_body(e, c2):
        exw[e, pl.ds(0, HD)] = exb[e, :]
        return c2

      lax.fori_loop(0, CHUNK, edge_body, 0)
      pltpu.sync_copy(exw, den_s.at[didx.at[0]], add=True)
      return carry

    lax.fori_loop(0, STEPS, step_body, 0)
    plsc.subcore_barrier()

    for q in range(10):
      rows = CHUNK if q < 9 else zrows - 9 * CHUNK
      off = sid * zrows + q * CHUNK
      pltpu.sync_copy(den_s.at[pl.ds(off, rows)], exw.at[pl.ds(0, rows)])
      pltpu.sync_copy(exw.at[pl.ds(0, rows)], den_out.at[cid].at[pl.ds(off, rows)])

  return k(d1d, exv, z128)


def _gelu(h):
  return 0.5 * h * (1.0 + lax.erf(h * 0.7071067811865476))


def _ln(h, g, b):
  mu = jnp.mean(h, axis=-1, keepdims=True)
  var = jnp.mean((h - mu) ** 2, axis=-1, keepdims=True)
  return (h - mu) * lax.rsqrt(var + 1e-5) * g + b


def _wspec(w):
  nd = w.ndim
  return pl.BlockSpec(w.shape, lambda i, _n=nd: (0,) * _n)


def _rowspec(cols):
  return pl.BlockSpec((BLK, cols), lambda i: (i, 0))


def _tc1_body(x_r, Win_r, bin_r, g_r, b_r, Wl_r, bl_r, Wr_r, br_r,
              h0_o, xl_o, xr_o):
  h = jnp.dot(x_r[...], Win_r[...], preferred_element_type=jnp.float32)
  h = h + bin_r[...]
  h = _gelu(_ln(h, g_r[...], b_r[...]))
  h0_o[...] = h
  xl_o[...] = jnp.dot(h, Wl_r[...], preferred_element_type=jnp.float32) + bl_r[...]
  xr_o[...] = jnp.dot(h, Wr_r[...], preferred_element_type=jnp.float32) + br_r[...]


def _tc1(x, p, Wl, bl, Wr, br):
  outs = [jax.ShapeDtypeStruct((N, HID), jnp.float32)] * 3
  ins = (x, p['W_in'], p['b_in'], p['ln1_g'], p['ln1_b'], Wl, bl, Wr, br)
  return pl.pallas_call(
      _tc1_body,
      grid=(N // BLK,),
      in_specs=[_rowspec(HID)] + [_wspec(w) for w in ins[1:]],
      out_specs=[_rowspec(HID)] * 3,
      out_shape=outs,
  )(*ins)


def _tc2_body(a0_r, a1_r, d0_r, d1_r, P_r, S_r, bias_r, g_r, b_r, hp_r,
              Wl_r, bl_r, Wr_r, br_r, h1_o, xl_o, xr_o):
  den = d0_r[...] + d1_r[...]
  recip = jnp.dot(1.0 / den, S_r[...], preferred_element_type=jnp.float32)
  a = jnp.dot(a0_r[...] + a1_r[...], P_r[...], preferred_element_type=jnp.float32)
  gacc = a * recip + bias_r[...]
  t = _gelu(_ln(gacc, g_r[...], b_r[...]))
  h1 = hp_r[...] + t
  h1_o[...] = h1
  xl_o[...] = jnp.dot(h1, Wl_r[...], preferred_element_type=jnp.float32) + bl_r[...]
  xr_o[...] = jnp.dot(h1, Wr_r[...], preferred_element_type=jnp.float32) + br_r[...]


def _tc2(acc, den, P128, S16, bias, ln_g, ln_b, h_prev, Wl, bl, Wr, br):
  outs = [jax.ShapeDtypeStruct((N, HID), jnp.float32)] * 3
  ins = (acc[0], acc[1], den[0], den[1], P128, S16, bias, ln_g, ln_b, h_prev,
         Wl, bl, Wr, br)
  in_specs = ([_rowspec(HID)] * 2 + [_rowspec(HD)] * 2 +
              [_wspec(P128), _wspec(S16), _wspec(bias), _wspec(ln_g),
               _wspec(ln_b), _rowspec(HID)] + [_wspec(w) for w in ins[10:]])
  return pl.pallas_call(
      _tc2_body,
      grid=(N // BLK,),
      in_specs=in_specs,
      out_specs=[_rowspec(HID)] * 3,
      out_shape=outs,
  )(*ins)


def _tc3_body(a0_r, a1_r, d0_r, d1_r, P_r, S_r, bias_r, g_r, b_r, hp_r,
              W1_r, b1_r, W2_r, b2_r, W3_r, b3_r, h_o, sc_o):
  den = d0_r[...] + d1_r[...]
  recip = jnp.dot(1.0 / den, S_r[...], preferred_element_type=jnp.float32)
  a = jnp.dot(a0_r[...] + a1_r[...], P_r[...], preferred_element_type=jnp.float32)
  gacc = a * recip + bias_r[...]
  t = _gelu(_ln(gacc, g_r[...], b_r[...]))
  h = hp_r[...] + t
  h_o[...] = h
  u = jnp.dot(h, W1_r[...], preferred_element_type=jnp.float32) + b1_r[...]
  u = u * jax.nn.sigmoid(u)
  v = jnp.dot(u, W2_r[...], preferred_element_type=jnp.float32) + b2_r[...]
  v = v * jax.nn.sigmoid(v)
  sc_o[...] = jnp.dot(v, W3_r[...], preferred_element_type=jnp.float32) + b3_r[...]


def _tc3(acc, den, P128, S16, bias, ln_g, ln_b, h_prev, p):
  outs = [jax.ShapeDtypeStruct((N, HID), jnp.float32),
          jax.ShapeDtypeStruct((N, 1), jnp.float32)]
  ins = (acc[0], acc[1], den[0], den[1], P128, S16, bias, ln_g, ln_b, h_prev,
         p['s_W1'], p['s_b1'], p['s_W2'], p['s_b2'], p['s_W3'], p['s_b3'])
  in_specs = ([_rowspec(HID)] * 2 + [_rowspec(HD)] * 2 +
              [_wspec(P128), _wspec(S16), _wspec(bias), _wspec(ln_g),
               _wspec(ln_b), _rowspec(HID)] + [_wspec(w) for w in ins[10:]])
  return pl.pallas_call(
      _tc3_body,
      grid=(N // BLK,),
      in_specs=in_specs,
      out_specs=[_rowspec(HID), pl.BlockSpec((BLK, 1), lambda i: (i, 0))],
      out_shape=outs,
  )(*ins)


def _gat_layer(h_tbl_l, h_tbl_r, s1d, d1d, att, z128, z16):
  acc, exv = _sc_gat_edges(h_tbl_l, h_tbl_r, s1d, d1d, att, z128)
  denw = _sc_gat_denom(d1d, exv, z128)[0]
  return acc, denw[:, :, :HD]


def kernel(x, edge_index, params):
  p = params

  loop = jnp.arange(N, dtype=jnp.int32)
  pad = jnp.full((EP - EB,), PAD_IDX, jnp.int32)
  s1d = jnp.concatenate([edge_index[0], loop, pad])
  d1d = jnp.concatenate([edge_index[1], loop, pad])

  z128 = jnp.zeros((ROWPAD, HID), jnp.float32)
  z16 = jnp.zeros((ROWPAD, HD), jnp.float32)
  rpad = jnp.zeros((ROWPAD - N, HID), jnp.float32)

  col = jnp.arange(HID)[None, :]
  row = jnp.arange(HD)[:, None]
  S16 = ((col // HD == row) & (row < HEADS)).astype(jnp.float32)

  f = jnp.arange(HID)
  jj, ll = f // HD, f % HD
  cch = 2 * jj + (ll >= 8)
  hh = jnp.where(ll < 8, ll, 15 - ll)
  perm = hh * HD + cch
  P128 = (jnp.arange(HID)[None, :] == perm[:, None]).astype(jnp.float32)

  def pw(g):
    return (g['Wl'][:, perm], g['bl'][perm], g['Wr'][:, perm], g['br'][perm],
            g['att'].reshape(HID)[perm].reshape(HEADS, HD))

  Wl1, bl1, Wr1, br1, att1 = pw(p['g1'])
  Wl2, bl2, Wr2, br2, att2 = pw(p['g2'])

  h0, xl1, xr1 = _tc1(x, p, Wl1, bl1, Wr1, br1)
  acc1, den1 = _gat_layer(
      jnp.concatenate([xl1, rpad]), jnp.concatenate([xr1, rpad]),
      s1d, d1d, att1, z128, z16)
  h1, xl2, xr2 = _tc2(acc1, den1, P128, S16, p['g1']['bias'],
                      p['ln2_g'], p['ln2_b'], h0, Wl2, bl2, Wr2, br2)
  acc2, den2 = _gat_layer(
      jnp.concatenate([xl2, rpad]), jnp.concatenate([xr2, rpad]),
      s1d, d1d, att2, z128, z16)
  h_out, scores = _tc3(acc2, den2, P128, S16, p['g2']['bias'],
                       p['ln3_g'], p['ln3_b'], h1, p)
  return scores.reshape(N), h_out
